# baseline (speedup 1.0000x reference)
import jax
import jax.numpy as jnp
from jax import lax
from jax.experimental import pallas as pl
from jax.experimental.pallas import tpu as pltpu
from jax.experimental.pallas import tpu_sc as plsc

N_USER = 10000
N_ITEM = 5000
N_ENT = 20000
D = 128
H = 32
E = 500000
B = 1024
NS = 16
K = 512
NCH = 63
EPT = NCH * K
EPAD = EPT * NS
R_E1 = 20480
R_U1 = 10240
SEC_E = 4 * R_E1
SEC_U = 4 * R_U1
ZR = 160
BT = B // NS
NV = H // 16
NB = 3


def _body(e0, u0, relf, kg_pack, ui_pack, users, pos, neg,
          scores, tab_e, tab_u,
          ent_acc, usr_acc, stage, rowbuf0, rowbuf1, rowbuf2,
          ebuf0, ebuf1, ebuf2, relv,
          ures, prow, tmp, sv, uidx, pidx,
          gsem0, gsem1, gsem2, ssem0, ssem1, ssem2,
          isem0, isem1, isem2):
    rowbufs = (rowbuf0, rowbuf1, rowbuf2)
    ebufs = (ebuf0, ebuf1, ebuf2)
    gsems = (gsem0, gsem1, gsem2)
    ssems = (ssem0, ssem1, ssem2)
    isems = (isem0, isem1, isem2)
    c = lax.axis_index("c")
    t = lax.axis_index("s")

    def adjust(idxref, nvec, off):
        def bd(i, _):
            sl = pl.ds(pl.multiple_of(i * 16, 16), 16)
            idxref[sl] = idxref[sl] + off
            return 0
        lax.fori_loop(0, nvec, bd, 0)

    def vadd_into(dst, src, n):
        def bd(j, _):
            for cc in range(NV):
                sl = pl.ds(cc * 16, 16)
                dst[j, sl] = dst[j, sl] + src[j, sl]
            return 0
        lax.fori_loop(0, n, bd, 0)

    def zero_stage():
        z = jnp.zeros((16,), jnp.float32)
        def bd(j, _):
            for cc in range(NV):
                stage[j, pl.ds(cc * 16, 16)] = z
            return 0
        lax.fori_loop(0, ZR, bd, 0)

    def zero_accs():
        for k in range(1280 // ZR):
            pltpu.sync_copy(stage, ent_acc.at[pl.ds(t * 1280 + k * ZR, ZR)])
        for k in range(640 // ZR):
            pltpu.sync_copy(stage, usr_acc.at[pl.ds(t * 640 + k * ZR, ZR)])

    def flush(acc_ref, nchunks, hbm_ref, hoff, rpt):
        for k in range(nchunks):
            pltpu.sync_copy(acc_ref.at[pl.ds(t * rpt + k * ZR, ZR)], stage)
            pltpu.sync_copy(stage, hbm_ref.at[pl.ds(hoff + t * rpt + k * ZR, ZR)])

    def edge_pass(src_ref, src_off, pack, srow, drow, use_rel, acc_ref):
        NGRP = NCH // NB

        def idx_issue(ch, b):
            pltpu.async_copy(pack.at[t * NCH + ch], ebufs[b], isems[b])

        def idx_wait(ch, b):
            pltpu.make_async_copy(
                pack.at[t * NCH + ch], ebufs[b], isems[b]).wait()

        def adjust_and_gather(b):
            @plsc.parallel_loop(0, K // 16, unroll=4)
            def adj(i):
                sl = pl.ds(pl.multiple_of(i * 16, 16), 16)
                ebufs[b][srow, sl] = ebufs[b][srow, sl] + src_off
            pltpu.async_copy(src_ref.at[ebufs[b].at[srow]], rowbufs[b], gsems[b])

        def process(b):
            pltpu.make_async_copy(
                src_ref.at[ebufs[b].at[srow]], rowbufs[b], gsems[b]).wait()
            @plsc.parallel_loop(0, K // 16, unroll=2)
            def inner(g2):
                b16 = pl.multiple_of(g2 * 16, 16)
                mv = ebufs[b][2, pl.ds(b16, 16)]
                for jj in range(16):
                    j = b16 + jj
                    if use_rel:
                        ty = mv[jj]
                        for cc in range(NV):
                            sl = pl.ds(cc * 16, 16)
                            rowbufs[b][j, sl] = rowbufs[b][j, sl] * relv[ty, sl]
                    else:
                        w = lax.bitcast_convert_type(mv[jj], jnp.float32)
                        for cc in range(NV):
                            sl = pl.ds(cc * 16, 16)
                            rowbufs[b][j, sl] = rowbufs[b][j, sl] * w
            pltpu.async_copy(
                rowbufs[b], acc_ref.at[ebufs[b].at[drow]], ssems[b], add=True)

        def scatter_wait(b):
            pltpu.make_async_copy(
                rowbufs[b], acc_ref.at[ebufs[b].at[drow]], ssems[b]).wait()

        pltpu.sync_copy(pack.at[t * NCH], ebufs[0])
        adjust_and_gather(0)
        idx_issue(1, 1)

        def outer(g, _):
            for b in range(NB):
                c3 = g * NB + b
                nxt, pb = (b + 1) % NB, (b + 2) % NB
                if b == NB - 1:
                    @pl.when(g < NGRP - 1)
                    def _():
                        idx_wait(c3 + 1, nxt)
                        adjust_and_gather(nxt)
                else:
                    idx_wait(c3 + 1, nxt)
                    adjust_and_gather(nxt)
                if b == 0:
                    @pl.when(g >= 1)
                    def _():
                        scatter_wait(pb)
                    idx_issue(c3 + 2, pb)
                else:
                    @pl.when(g < NGRP - 1)
                    def _():
                        scatter_wait(pb)
                        idx_issue(c3 + 2, pb)
                process(b)
            return 0
        lax.fori_loop(0, NGRP, outer, 0)
        for b in range(NB):
            scatter_wait(b)

    def dots(aref, bref, out_ref):
        lane = lax.iota(jnp.int32, 16)
        def bd(g, _):
            b16 = pl.multiple_of(g * 16, 16)
            sv_vec = jnp.zeros((16,), jnp.float32)
            for jj in range(16):
                j = b16 + jj
                acc = aref[j, pl.ds(0, 16)] * bref[j, pl.ds(0, 16)]
                for cc in range(1, NV):
                    sl = pl.ds(cc * 16, 16)
                    acc = acc + aref[j, sl] * bref[j, sl]
                s = acc[0]
                for l in range(1, 16):
                    s = s + acc[l]
                sv_vec = jnp.where(lane == jj, s, sv_vec)
            out_ref[pl.ds(b16, 16)] = sv_vec
            return 0
        lax.fori_loop(0, BT // 16, bd, 0)

    bsl = pl.ds(t * BT, BT)

    def copyin(src_hbm, dst_hbm, rows_per_tile):
        def bd(k, _):
            base = pl.multiple_of(t * rows_per_tile + k * ZR, ZR)
            pltpu.sync_copy(src_hbm.at[pl.ds(base, ZR)], stage)
            pltpu.sync_copy(stage, dst_hbm.at[pl.ds(base, ZR)])
            return 0
        lax.fori_loop(0, rows_per_tile // ZR, bd, 0)
    copyin(e0, tab_e, SEC_E // NS)
    copyin(u0, tab_u, SEC_U // NS)

    def quarter(qp, _):
        vq = c * 2 + qp
        pltpu.sync_copy(relf.at[pl.ds(vq * 8, 8)], relv)

        zero_stage()
        zero_accs()
        plsc.subcore_barrier()

        def hop(h, _):
            e_off = h * SEC_E + vq * R_E1
            u_off = h * SEC_U + vq * R_U1
            edge_pass(tab_e, e_off, kg_pack, 0, 1, True, ent_acc)
            edge_pass(tab_e, e_off, ui_pack, 0, 1, False, usr_acc)
            edge_pass(tab_u, u_off, ui_pack, 1, 0, False, ent_acc)
            plsc.subcore_barrier()

            @pl.when(h == 0)
            def _():
                flush(ent_acc, 1280 // ZR, tab_e, SEC_E + vq * R_E1, 1280)
                flush(usr_acc, 640 // ZR, tab_u, SEC_U + vq * R_U1, 640)
                zero_stage()
                zero_accs()
            plsc.subcore_barrier()
            return 0
        lax.fori_loop(0, 2, hop, 0)

        pltpu.sync_copy(users.at[bsl], uidx)
        pltpu.sync_copy(usr_acc.at[uidx], ures)
        adjust(uidx, BT // 16, vq * R_U1)
        pltpu.sync_copy(tab_u.at[uidx], tmp)
        vadd_into(ures, tmp, BT)
        adjust(uidx, BT // 16, SEC_U)
        pltpu.sync_copy(tab_u.at[uidx], tmp)
        vadd_into(ures, tmp, BT)

        def item_scores(idx_hbm, out_off):
            pltpu.sync_copy(idx_hbm.at[bsl], pidx)
            pltpu.sync_copy(ent_acc.at[pidx], prow)
            adjust(pidx, BT // 16, vq * R_E1)
            pltpu.sync_copy(tab_e.at[pidx], tmp)
            vadd_into(prow, tmp, BT)
            adjust(pidx, BT // 16, SEC_E)
            pltpu.sync_copy(tab_e.at[pidx], tmp)
            vadd_into(prow, tmp, BT)
            dots(ures, prow, sv)
            pltpu.sync_copy(sv, scores.at[pl.ds(out_off + t * BT, BT)])
        item_scores(pos, vq * (2 * B))
        item_scores(neg, vq * (2 * B) + B)

        plsc.subcore_barrier()
        return 0
    lax.fori_loop(0, 2, quarter, 0)


def _pad_rows(x, rows):
    return jnp.concatenate(
        [x, jnp.zeros((rows - x.shape[0], x.shape[1]), x.dtype)], axis=0)


def _quarters(x, rows):
    return jnp.concatenate(
        [_pad_rows(x[:, q * H:(q + 1) * H], rows) for q in range(4)], axis=0)


def kernel(all_embed, rel_emb, rates_param, inter_edge_w, users, pos, neg,
           edge_index, edge_type, inter_user, inter_item):
    del rates_param
    i32 = jnp.int32
    npad = EPAD - E

    e0 = _quarters(all_embed[N_USER:], R_E1)
    u0 = _quarters(all_embed[:N_USER], R_U1)
    relf = _quarters(rel_emb, 8)

    def padi(x, pad_vals):
        return jnp.concatenate([x.astype(i32), pad_vals.astype(i32)])

    ent_pad = N_ENT + (jnp.arange(npad, dtype=i32) % (R_E1 - N_ENT))
    usr_pad = N_USER + (jnp.arange(npad, dtype=i32) % (R_U1 - N_USER))
    csrc = jnp.full((npad,), N_ENT, i32)
    usrc = jnp.full((npad,), N_USER, i32)

    headp = padi(edge_index[0], ent_pad)
    tailp = padi(edge_index[1], csrc)
    typep = padi(edge_type, jnp.zeros((npad,), i32))
    iup = padi(inter_user, usr_pad)
    iip = padi(inter_item, ent_pad)

    def pack3(a, b, cw):
        arr = jnp.stack([a, b, cw], axis=0).reshape(3, NS * NCH, K)
        return arr.transpose(1, 0, 2)

    kg_pack = pack3(tailp, headp, typep)
    wbits = lax.bitcast_convert_type(
        jnp.concatenate([inter_edge_w, jnp.zeros((npad,), jnp.float32)]), i32)
    ui_pack = pack3(iip, iup, wbits)

    f32 = jnp.float32
    run = pl.kernel(
        _body,
        out_type=(
            jax.ShapeDtypeStruct((8 * B,), f32),
            jax.ShapeDtypeStruct((2 * SEC_E, H), f32),
            jax.ShapeDtypeStruct((2 * SEC_U, H), f32),
        ),
        mesh=plsc.VectorSubcoreMesh(core_axis_name="c", subcore_axis_name="s"),
        compiler_params=pltpu.CompilerParams(use_tc_tiling_on_sc=False),
        scratch_types=(
            pltpu.VMEM_SHARED((R_E1, H), f32),
            pltpu.VMEM_SHARED((R_U1, H), f32),
            pltpu.VMEM((ZR, H), f32),
            pltpu.VMEM((K, H), f32),
            pltpu.VMEM((K, H), f32),
            pltpu.VMEM((K, H), f32),
            pltpu.VMEM((3, K), i32),
            pltpu.VMEM((3, K), i32),
            pltpu.VMEM((3, K), i32),
            pltpu.VMEM((8, H), f32),
            pltpu.VMEM((BT, H), f32),
            pltpu.VMEM((BT, H), f32),
            pltpu.VMEM((BT, H), f32),
            pltpu.VMEM((BT,), f32),
            pltpu.VMEM((BT,), i32),
            pltpu.VMEM((BT,), i32),
            pltpu.SemaphoreType.DMA,
            pltpu.SemaphoreType.DMA,
            pltpu.SemaphoreType.DMA,
            pltpu.SemaphoreType.DMA,
            pltpu.SemaphoreType.DMA,
            pltpu.SemaphoreType.DMA,
            pltpu.SemaphoreType.DMA,
            pltpu.SemaphoreType.DMA,
            pltpu.SemaphoreType.DMA,
        ),
    )
    scores8, _, _ = run(e0, u0, relf, kg_pack, ui_pack,
                        users.astype(i32), pos.astype(i32), neg.astype(i32))
    s = scores8.reshape(4, 2, B)
    return s[0] + s[1] + s[2] + s[3]


# speedup vs baseline: 1.0873x; 1.0873x over previous
import jax
import jax.numpy as jnp
from jax import lax
from jax.experimental import pallas as pl
from jax.experimental.pallas import tpu as pltpu
from jax.experimental.pallas import tpu_sc as plsc

N_USER = 10000
N_ITEM = 5000
N_ENT = 20000
D = 128
H = 32
E = 500000
B = 1024
NS = 16
K = 512
NCH = 63
EPT = NCH * K
EPAD = EPT * NS
R_E0 = 20016
R_E1 = 20480
R_U0 = 10016
R_U1 = 10240
ZR = 160
BT = B // NS
NV = H // 16
NB = 3


def _body(e0, u0, relf, kg_pack, ui_pack, users, pos, neg,
          scores, e1, u1,
          ent_acc, usr_acc, stage, rowbuf0, rowbuf1, rowbuf2,
          ebuf0, ebuf1, ebuf2, dbuf0, dbuf1, dbuf2, relv,
          ures, prow, tmp, sv, uidx, pidx,
          gsem0, gsem1, gsem2, ssem0, ssem1, ssem2,
          isem0, isem1, isem2):
    rowbufs = (rowbuf0, rowbuf1, rowbuf2)
    ebufs = (ebuf0, ebuf1, ebuf2)
    dbufs = (dbuf0, dbuf1, dbuf2)
    gsems = (gsem0, gsem1, gsem2)
    ssems = (ssem0, ssem1, ssem2)
    isems = (isem0, isem1, isem2)
    c = lax.axis_index("c")
    t = lax.axis_index("s")

    def adjust(idxref, nvec, off):
        def bd(i, _):
            sl = pl.ds(pl.multiple_of(i * 16, 16), 16)
            idxref[sl] = idxref[sl] + off
            return 0
        lax.fori_loop(0, nvec, bd, 0)

    def vadd_into(dst, src, n):
        def bd(j, _):
            for cc in range(NV):
                sl = pl.ds(cc * 16, 16)
                dst[j, sl] = dst[j, sl] + src[j, sl]
            return 0
        lax.fori_loop(0, n, bd, 0)

    def zero_stage():
        z = jnp.zeros((16,), jnp.float32)
        def bd(j, _):
            for cc in range(NV):
                stage[j, pl.ds(cc * 16, 16)] = z
            return 0
        lax.fori_loop(0, ZR, bd, 0)

    def zero_accs():
        for k in range(1280 // ZR):
            pltpu.sync_copy(stage, ent_acc.at[pl.ds(t * 1280 + k * ZR, ZR)])
        for k in range(640 // ZR):
            pltpu.sync_copy(stage, usr_acc.at[pl.ds(t * 640 + k * ZR, ZR)])

    def flush(acc_ref, nchunks, hbm_ref, hoff, rpt):
        for k in range(nchunks):
            pltpu.sync_copy(acc_ref.at[pl.ds(t * rpt + k * ZR, ZR)], stage)
            pltpu.sync_copy(stage, hbm_ref.at[pl.ds(hoff + t * rpt + k * ZR, ZR)])

    def edge_pass(src_ref, src_off, pack, srow, drow, use_rel, acc_ref):
        NGRP = NCH // NB

        def idx_issue(ch, b):
            pltpu.async_copy(pack.at[t * NCH + ch], ebufs[b], isems[b])

        def idx_wait(ch, b):
            pltpu.make_async_copy(
                pack.at[t * NCH + ch], ebufs[b], isems[b]).wait()

        def adjust_and_gather(b):
            @plsc.parallel_loop(0, K // 16, unroll=4)
            def adj(i):
                sl = pl.ds(pl.multiple_of(i * 16, 16), 16)
                ebufs[b][srow, sl] = ebufs[b][srow, sl] + src_off
                dbufs[b][sl] = ebufs[b][drow, sl]
            pltpu.async_copy(src_ref.at[ebufs[b].at[srow]], rowbufs[b], gsems[b])

        def process(b):
            pltpu.make_async_copy(
                src_ref.at[ebufs[b].at[srow]], rowbufs[b], gsems[b]).wait()
            @plsc.parallel_loop(0, K // 16, unroll=1)
            def inner(g2):
                b16 = pl.multiple_of(g2 * 16, 16)
                mv = ebufs[b][2, pl.ds(b16, 16)]
                for jj in range(16):
                    j = b16 + jj
                    if use_rel:
                        ty = mv[jj]
                        for cc in range(NV):
                            sl = pl.ds(cc * 16, 16)
                            rowbufs[b][j, sl] = rowbufs[b][j, sl] * relv[ty, sl]
                    else:
                        w = lax.bitcast_convert_type(mv[jj], jnp.float32)
                        for cc in range(NV):
                            sl = pl.ds(cc * 16, 16)
                            rowbufs[b][j, sl] = rowbufs[b][j, sl] * w
            pltpu.async_copy(
                rowbufs[b], acc_ref.at[dbufs[b]], ssems[b], add=True)

        def scatter_wait(b):
            pltpu.make_async_copy(
                rowbufs[b], acc_ref.at[dbufs[b]], ssems[b]).wait()

        pltpu.sync_copy(pack.at[t * NCH], ebufs[0])
        adjust_and_gather(0)
        idx_issue(1, 1)

        def outer(g, _):
            for b in range(NB):
                c3 = g * NB + b
                nxt, pb = (b + 1) % NB, (b + 2) % NB
                def stage_c():
                    idx_wait(c3 + 1, nxt)
                    adjust_and_gather(nxt)
                if b == NB - 1:
                    @pl.when(g < NGRP - 1)
                    def _():
                        scatter_wait(nxt)
                        stage_c()
                else:
                    if b == 0 or b == 1:
                        @pl.when(g >= 1)
                        def _():
                            scatter_wait(nxt)
                    stage_c()
                if b == 0:
                    idx_issue(c3 + 2, pb)
                else:
                    @pl.when(g < NGRP - 1)
                    def _():
                        idx_issue(c3 + 2, pb)
                process(b)
            return 0
        lax.fori_loop(0, NGRP, outer, 0)
        for b in range(NB):
            scatter_wait(b)

    def dots(aref, bref, out_ref):
        lane = lax.iota(jnp.int32, 16)
        def bd(g, _):
            b16 = pl.multiple_of(g * 16, 16)
            sv_vec = jnp.zeros((16,), jnp.float32)
            for jj in range(16):
                j = b16 + jj
                acc = aref[j, pl.ds(0, 16)] * bref[j, pl.ds(0, 16)]
                for cc in range(1, NV):
                    sl = pl.ds(cc * 16, 16)
                    acc = acc + aref[j, sl] * bref[j, sl]
                s = acc[0]
                for l in range(1, 16):
                    s = s + acc[l]
                sv_vec = jnp.where(lane == jj, s, sv_vec)
            out_ref[pl.ds(b16, 16)] = sv_vec
            return 0
        lax.fori_loop(0, BT // 16, bd, 0)

    bsl = pl.ds(t * BT, BT)

    def quarter(qp, _):
        vq = c * 2 + qp
        pltpu.sync_copy(relf.at[pl.ds(vq * 8, 8)], relv)

        zero_stage()
        zero_accs()
        plsc.subcore_barrier()

        edge_pass(e0, vq * R_E0, kg_pack, 0, 1, True, ent_acc)
        edge_pass(e0, vq * R_E0, ui_pack, 0, 1, False, usr_acc)
        edge_pass(u0, vq * R_U0, ui_pack, 1, 0, False, ent_acc)
        plsc.subcore_barrier()

        flush(ent_acc, 1280 // ZR, e1, vq * R_E1, 1280)
        flush(usr_acc, 640 // ZR, u1, vq * R_U1, 640)
        zero_stage()
        zero_accs()
        plsc.subcore_barrier()

        edge_pass(e1, vq * R_E1, kg_pack, 0, 1, True, ent_acc)
        edge_pass(e1, vq * R_E1, ui_pack, 0, 1, False, usr_acc)
        edge_pass(u1, vq * R_U1, ui_pack, 1, 0, False, ent_acc)
        plsc.subcore_barrier()

        pltpu.sync_copy(users.at[bsl], uidx)
        pltpu.sync_copy(usr_acc.at[uidx], ures)
        adjust(uidx, BT // 16, vq * R_U0)
        pltpu.sync_copy(u0.at[uidx], tmp)
        vadd_into(ures, tmp, BT)
        adjust(uidx, BT // 16, vq * (R_U1 - R_U0))
        pltpu.sync_copy(u1.at[uidx], tmp)
        vadd_into(ures, tmp, BT)

        pltpu.sync_copy(pos.at[bsl], pidx)
        pltpu.sync_copy(ent_acc.at[pidx], prow)
        adjust(pidx, BT // 16, vq * R_E0)
        pltpu.sync_copy(e0.at[pidx], tmp)
        vadd_into(prow, tmp, BT)
        adjust(pidx, BT // 16, vq * (R_E1 - R_E0))
        pltpu.sync_copy(e1.at[pidx], tmp)
        vadd_into(prow, tmp, BT)
        dots(ures, prow, sv)
        pltpu.sync_copy(sv, scores.at[pl.ds(vq * (2 * B) + t * BT, BT)])

        pltpu.sync_copy(neg.at[bsl], pidx)
        pltpu.sync_copy(ent_acc.at[pidx], prow)
        adjust(pidx, BT // 16, vq * R_E0)
        pltpu.sync_copy(e0.at[pidx], tmp)
        vadd_into(prow, tmp, BT)
        adjust(pidx, BT // 16, vq * (R_E1 - R_E0))
        pltpu.sync_copy(e1.at[pidx], tmp)
        vadd_into(prow, tmp, BT)
        dots(ures, prow, sv)
        pltpu.sync_copy(sv, scores.at[pl.ds(vq * (2 * B) + B + t * BT, BT)])

        plsc.subcore_barrier()
        return 0


    lax.fori_loop(0, 2, quarter, 0)


def _pad_rows(x, rows):
    return jnp.concatenate(
        [x, jnp.zeros((rows - x.shape[0], x.shape[1]), x.dtype)], axis=0)


def _quarters(x, rows):
    return jnp.concatenate(
        [_pad_rows(x[:, q * H:(q + 1) * H], rows) for q in range(4)], axis=0)


def kernel(all_embed, rel_emb, rates_param, inter_edge_w, users, pos, neg,
           edge_index, edge_type, inter_user, inter_item):
    del rates_param
    i32 = jnp.int32
    npad = EPAD - E

    e0 = _quarters(all_embed[N_USER:], R_E0)
    u0 = _quarters(all_embed[:N_USER], R_U0)
    relf = _quarters(rel_emb, 8)

    def padi(x, pad_vals):
        return jnp.concatenate([x.astype(i32), pad_vals.astype(i32)])

    ent_pad = N_ENT + (jnp.arange(npad, dtype=i32) % (R_E1 - N_ENT))
    usr_pad = N_USER + (jnp.arange(npad, dtype=i32) % (R_U1 - N_USER))
    csrc = jnp.full((npad,), N_ENT, i32)
    usrc = jnp.full((npad,), N_USER, i32)

    headp = padi(edge_index[0], ent_pad)
    tailp = padi(edge_index[1], csrc)
    typep = padi(edge_type, jnp.zeros((npad,), i32))
    iup = padi(inter_user, usr_pad)
    iip = padi(inter_item, ent_pad)

    def pack3(a, b, cw):
        arr = jnp.stack([a, b, cw], axis=0).reshape(3, NS * NCH, K)
        return arr.transpose(1, 0, 2)

    kg_pack = pack3(tailp, headp, typep)
    wbits = lax.bitcast_convert_type(
        jnp.concatenate([inter_edge_w, jnp.zeros((npad,), jnp.float32)]), i32)
    ui_pack = pack3(iip, iup, wbits)

    f32 = jnp.float32
    run = pl.kernel(
        _body,
        out_type=(
            jax.ShapeDtypeStruct((8 * B,), f32),
            jax.ShapeDtypeStruct((4 * R_E1, H), f32),
            jax.ShapeDtypeStruct((4 * R_U1, H), f32),
        ),
        mesh=plsc.VectorSubcoreMesh(core_axis_name="c", subcore_axis_name="s"),
        compiler_params=pltpu.CompilerParams(use_tc_tiling_on_sc=False),
        scratch_types=(
            pltpu.VMEM_SHARED((R_E1, H), f32),
            pltpu.VMEM_SHARED((R_U1, H), f32),
            pltpu.VMEM((ZR, H), f32),
            pltpu.VMEM((K, H), f32),
            pltpu.VMEM((K, H), f32),
            pltpu.VMEM((K, H), f32),
            pltpu.VMEM((3, K), i32),
            pltpu.VMEM((3, K), i32),
            pltpu.VMEM((3, K), i32),
            pltpu.VMEM((K,), i32),
            pltpu.VMEM((K,), i32),
            pltpu.VMEM((K,), i32),
            pltpu.VMEM((8, H), f32),
            pltpu.VMEM((BT, H), f32),
            pltpu.VMEM((BT, H), f32),
            pltpu.VMEM((BT, H), f32),
            pltpu.VMEM((BT,), f32),
            pltpu.VMEM((BT,), i32),
            pltpu.VMEM((BT,), i32),
            pltpu.SemaphoreType.DMA,
            pltpu.SemaphoreType.DMA,
            pltpu.SemaphoreType.DMA,
            pltpu.SemaphoreType.DMA,
            pltpu.SemaphoreType.DMA,
            pltpu.SemaphoreType.DMA,
            pltpu.SemaphoreType.DMA,
            pltpu.SemaphoreType.DMA,
            pltpu.SemaphoreType.DMA,
        ),
    )
    scores8, _, _ = run(e0, u0, relf, kg_pack, ui_pack,
                        users.astype(i32), pos.astype(i32), neg.astype(i32))
    s = scores8.reshape(4, 2, B)
    return s[0] + s[1] + s[2] + s[3]
